# baseline (speedup 1.0000x reference)
import functools

import jax
import jax.numpy as jnp
from jax import lax
from jax.experimental import pallas as pl
from jax.experimental.pallas import tpu as pltpu
from jax.experimental.pallas import tpu_sc as plsc

_NUM_CORES = 2
_NUM_SUBCORES = 16
_NW = _NUM_CORES * _NUM_SUBCORES

_BATCH = 16384
_DIM = 32
_ZONES = 9
_BPW = _BATCH // _NW
_CHUNKS = _BPW // 16


@functools.partial(
    pl.kernel,
    out_type=jax.ShapeDtypeStruct((_DIM, _BATCH), jnp.float32),
    mesh=plsc.VectorSubcoreMesh(
        core_axis_name="c",
        subcore_axis_name="s",
        num_cores=_NUM_CORES,
        num_subcores=_NUM_SUBCORES,
    ),
    scratch_types=[
        pltpu.VMEM((_BPW,), jnp.int32),
        pltpu.VMEM((_ZONES, _DIM), jnp.float32),
        pltpu.VMEM((_DIM, 16), jnp.float32),
        pltpu.VMEM((_DIM, _BPW), jnp.float32),
    ],
    compiler_params=pltpu.CompilerParams(
        needs_layout_passes=False
    ),
)
def _gather_kernel(idx_hbm, table_hbm, out_hbm, idx_v, table_v, ttv, rows_v):
    wid = lax.axis_index("s") * _NUM_CORES + lax.axis_index("c")
    base = wid * _BPW
    pltpu.sync_copy(idx_hbm.at[pl.ds(base, _BPW)], idx_v)
    pltpu.sync_copy(table_hbm, table_v)
    lane = lax.iota(jnp.int32, 16)

    @plsc.parallel_loop(0, _ZONES * (_DIM // 16), 1, unroll=2)
    def setup(t):
        r = t // (_DIM // 16)
        h = t % (_DIM // 16)
        vals = table_v[r, pl.ds(h * 16, 16)]
        plsc.store_scatter(
            ttv, [lane + h * 16, jnp.full((16,), 0, jnp.int32) + r], vals
        )

    @plsc.parallel_loop(0, _CHUNKS // 4, 1, unroll=1)
    def body(i):
        ridx = [idx_v[pl.ds(i * 64 + 16 * k, 16)] for k in range(4)]

        @plsc.parallel_loop(0, _DIM, 1, unroll=4)
        def cols(d):
            tcol = ttv[d]
            for k in range(4):
                rows_v[d, pl.ds(i * 64 + 16 * k, 16)] = tcol.at[ridx[k]].get(
                    mode="promise_in_bounds"
                )

    pltpu.sync_copy(rows_v, out_hbm.at[:, pl.ds(base, _BPW)])


def kernel(zone_idx, embedding_table):
    out_t = _gather_kernel(zone_idx.astype(jnp.int32), embedding_table)
    return out_t.T
